# baseline (speedup 1.0000x reference)
import functools

import jax
import jax.numpy as jnp
from jax import lax
from jax.experimental import pallas as pl
from jax.experimental.pallas import tpu as pltpu
from jax.experimental.pallas import tpu_sc as plsc

_N = 100000
_D = 128
_NC = 2
_NS = 16
_NW = _NC * _NS
_CHUNK = 200
_NCHUNK = _N // _CHUNK
_FULL = _NCHUNK // _NW
_EXTRA = _NCHUNK - _FULL * _NW


def _sc_body(feat, w0_hbm, w1_hbm, out, bufs, w0v, w1v, sems):
    wid = lax.axis_index("s") * _NC + lax.axis_index("c")

    pltpu.sync_copy(w0_hbm, w0v)
    pltpu.sync_copy(w1_hbm, w1v)
    w0r = [w0v[pl.ds(j * 16, 16)] for j in range(8)]
    w1r = [w1v[pl.ds(j * 16, 16)] for j in range(8)]

    def base(k):
        return (k * _NW + wid) * _CHUNK

    def start_in(slot, row):
        pltpu.make_async_copy(
            feat.at[pl.ds(row, _CHUNK)], bufs.at[slot], sems.at[slot]).start()

    def wait_in(slot, row):
        pltpu.make_async_copy(
            feat.at[pl.ds(row, _CHUNK)], bufs.at[slot], sems.at[slot]).wait()

    def start_out(slot, row):
        pltpu.make_async_copy(
            bufs.at[slot], out.at[pl.ds(row, _CHUNK)],
            sems.at[2 + slot]).start()

    def wait_out(slot, row):
        pltpu.make_async_copy(
            bufs.at[slot], out.at[pl.ds(row, _CHUNK)],
            sems.at[2 + slot]).wait()

    def compute(slot):
        def row_body(i, carry):
            for j in range(8):
                sl = pl.ds(j * 16, 16)
                x = bufs[slot, i, sl]
                bufs[slot, i, sl] = jnp.maximum(x * w0r[j], 0.0) * w1r[j]
            return carry
        lax.fori_loop(0, _CHUNK, row_body, 0)

    start_in(0, base(0))
    start_in(1, base(1))
    for k in range(_FULL):
        slot = k % 2
        wait_in(slot, base(k))
        compute(slot)
        start_out(slot, base(k))
        if k + 2 < _FULL:
            wait_out(slot, base(k))
            start_in(slot, base(k + 2))

    wait_out(_FULL % 2, base(_FULL - 1))
    wait_out((_FULL - 1) % 2, base(_FULL - 2))

    @pl.when(wid < _EXTRA)
    def _():
        row = (_FULL * _NW + wid) * _CHUNK
        start_in(0, row)
        wait_in(0, row)
        compute(0)
        start_out(0, row)
        wait_out(0, row)


def kernel(features, w0, w1):
    n, d = features.shape
    mesh = plsc.VectorSubcoreMesh(core_axis_name="c", subcore_axis_name="s")
    sc_call = functools.partial(
        pl.kernel,
        mesh=mesh,
        out_type=jax.ShapeDtypeStruct((n, d), jnp.float32),
        scratch_types=[
            pltpu.VMEM((2, _CHUNK, _D), jnp.float32),
            pltpu.VMEM((_D,), jnp.float32),
            pltpu.VMEM((_D,), jnp.float32),
            pltpu.SemaphoreType.DMA((4,)),
        ],
    )(_sc_body)
    return sc_call(features, w0, w1)


# speedup vs baseline: 2.0280x; 2.0280x over previous
import jax
import jax.numpy as jnp
from jax.experimental import pallas as pl
from jax.experimental.pallas import tpu as pltpu

_BLOCK_ROWS = 25000


def _att_kernel(x_ref, w0_ref, w1_ref, o_ref):
    o_ref[...] = jnp.maximum(x_ref[...] * w0_ref[...], 0.0) * w1_ref[...]


def kernel(features, w0, w1):
    n, d = features.shape
    return pl.pallas_call(
        _att_kernel,
        grid=(n // _BLOCK_ROWS,),
        in_specs=[
            pl.BlockSpec((_BLOCK_ROWS, d), lambda i: (i, 0)),
            pl.BlockSpec((1, d), lambda i: (0, 0)),
            pl.BlockSpec((1, d), lambda i: (0, 0)),
        ],
        out_specs=pl.BlockSpec((_BLOCK_ROWS, d), lambda i: (i, 0)),
        out_shape=jax.ShapeDtypeStruct((n, d), features.dtype),
        compiler_params=pltpu.CompilerParams(
            vmem_limit_bytes=60 * 1024 * 1024,
        ),
    )(features, w0.reshape(1, d), w1.reshape(1, d))
